# baseline (speedup 1.0000x reference)
import functools

import jax
import jax.numpy as jnp
from jax import lax
from jax.experimental import pallas as pl
from jax.experimental.pallas import tpu as pltpu
from jax.experimental.pallas import tpu_sc as plsc

NC, NS, L = 2, 16, 16
NW = NC * NS
BLEND = 0.5
NEG_SLOPE = 0.2
CH = 64


def _dense_body(x_ref, w_ref, asrc_ref, adst_ref,
                h_ref, ssrc_ref, sdst_ref, cvec_ref):
    h = jnp.dot(x_ref[...], w_ref[...], preferred_element_type=jnp.float32)
    h_ref[...] = h
    ssrc = jnp.sum(h * asrc_ref[...][None, :], axis=1, keepdims=True)
    sdst = jnp.sum(h * adst_ref[...][None, :], axis=1, keepdims=True)
    ssrc_ref[...] = ssrc
    sdst_ref[...] = sdst
    t = jnp.max(ssrc) + jnp.max(sdst)
    c = jnp.where(t > 0, t, NEG_SLOPE * t)
    cvec_ref[...] = jnp.full((1, L), c, jnp.float32)


def _blend_dense_body(x0_ref, p0_ref, p1_ref, w_ref, asrc_ref, adst_ref,
                      xn_ref, h_ref, ssrc_ref, sdst_ref, cvec_ref):
    xn = BLEND * (p0_ref[...] + p1_ref[...]) + (1.0 - BLEND) * x0_ref[...]
    xn_ref[...] = xn
    h = jnp.dot(xn, w_ref[...], preferred_element_type=jnp.float32)
    h_ref[...] = h
    ssrc = jnp.sum(h * asrc_ref[...][None, :], axis=1, keepdims=True)
    sdst = jnp.sum(h * adst_ref[...][None, :], axis=1, keepdims=True)
    ssrc_ref[...] = ssrc
    sdst_ref[...] = sdst
    t = jnp.max(ssrc) + jnp.max(sdst)
    c = jnp.where(t > 0, t, NEG_SLOPE * t)
    cvec_ref[...] = jnp.full((1, L), c, jnp.float32)


def _final_body(x0_ref, p0_ref, p1_ref, fcw_ref, fcb_ref, out_ref, x3_ref):
    x3 = BLEND * (p0_ref[...] + p1_ref[...]) + (1.0 - BLEND) * x0_ref[...]
    x3_ref[...] = x3
    out_ref[...] = (jnp.dot(x3, fcw_ref[...], preferred_element_type=jnp.float32)
                    + fcb_ref[...][None, :])


def _rdenom_body(dpart_ref, rd_ref):
    s = jnp.sum(dpart_ref[...], axis=0, keepdims=True)
    rd_ref[...] = 1.0 / (s + 1e-16)


def _make_sc_kernels(N, D, E, E_pad):
    EW = E_pad // NW
    G = EW // L
    NCH = EW // CH
    RPT = N // NS

    mesh = plsc.VectorSubcoreMesh(core_axis_name="c", subcore_axis_name="s",
                                  num_cores=NC, num_subcores=NS)

    @functools.partial(
        pl.kernel,
        out_type=(jax.ShapeDtypeStruct((NW, N), jnp.float32),
                  jax.ShapeDtypeStruct((E_pad,), jnp.float32)),
        mesh=mesh,
        compiler_params=pltpu.CompilerParams(use_tc_tiling_on_sc=False, needs_layout_passes=False),
        scratch_types=[
            pltpu.VMEM((EW,), jnp.int32),
            pltpu.VMEM((EW,), jnp.int32),
            pltpu.VMEM((N,), jnp.float32),
            pltpu.VMEM((N,), jnp.float32),
            pltpu.VMEM((N,), jnp.float32),
            pltpu.VMEM((EW,), jnp.float32),
            pltpu.VMEM((L,), jnp.float32),
        ],
    )
    def sc_pass1(src_hbm, dst_hbm, ssrc_hbm, sdst_hbm, cvec_hbm,
                 dpart_hbm, ex_hbm,
                 src_v, dst_v, ssrc_v, sdst_v, den_v, ex_v, cvec_v):
        cid = lax.axis_index("c")
        sid = lax.axis_index("s")
        wid = sid * NC + cid
        base = wid * EW
        pltpu.sync_copy(src_hbm.at[pl.ds(base, EW)], src_v)
        pltpu.sync_copy(dst_hbm.at[pl.ds(base, EW)], dst_v)
        pltpu.sync_copy(ssrc_hbm, ssrc_v)
        pltpu.sync_copy(sdst_hbm, sdst_v)
        pltpu.sync_copy(cvec_hbm, cvec_v)
        cv = cvec_v[...]

        def zbody(i, _):
            den_v[pl.ds(i * L, L)] = jnp.zeros((L,), jnp.float32)
            return 0
        lax.fori_loop(0, N // L, zbody, 0)

        lanes = lax.iota(jnp.int32, L)

        def gbody(g, _):
            off = g * L
            si = src_v[pl.ds(off, L)]
            di = dst_v[pl.ds(off, L)]
            a = plsc.load_gather(ssrc_v, [si])
            b = plsc.load_gather(sdst_v, [di])
            t = a + b
            t = jnp.where(t > 0, t, NEG_SLOPE * t)
            ex = jnp.exp(t - cv)
            gid = base + off + lanes
            ex = jnp.where(gid < E, ex, 0.0)
            ex_v[pl.ds(off, L)] = ex
            plsc.addupdate_scatter(den_v, [di], ex)
            return 0
        lax.fori_loop(0, G, gbody, 0)

        pltpu.sync_copy(den_v, dpart_hbm.at[wid])
        pltpu.sync_copy(ex_v, ex_hbm.at[pl.ds(base, EW)])

    @functools.partial(
        pl.kernel,
        out_type=jax.ShapeDtypeStruct((E_pad,), jnp.float32),
        mesh=mesh,
        compiler_params=pltpu.CompilerParams(use_tc_tiling_on_sc=False, needs_layout_passes=False),
        scratch_types=[
            pltpu.VMEM((EW,), jnp.int32),
            pltpu.VMEM((EW,), jnp.float32),
            pltpu.VMEM((N,), jnp.float32),
            pltpu.VMEM((EW,), jnp.float32),
        ],
    )
    def sc_coef(dst_hbm, ex_hbm, rd_hbm, cf_hbm, dst_v, ex_v, rd_v, cf_v):
        cid = lax.axis_index("c")
        sid = lax.axis_index("s")
        wid = sid * NC + cid
        base = wid * EW
        pltpu.sync_copy(dst_hbm.at[pl.ds(base, EW)], dst_v)
        pltpu.sync_copy(ex_hbm.at[pl.ds(base, EW)], ex_v)
        pltpu.sync_copy(rd_hbm, rd_v)

        def gbody(g, _):
            off = g * L
            di = dst_v[pl.ds(off, L)]
            cf_v[pl.ds(off, L)] = ex_v[pl.ds(off, L)] * plsc.load_gather(rd_v, [di])
            return 0
        lax.fori_loop(0, G, gbody, 0)
        pltpu.sync_copy(cf_v, cf_hbm.at[pl.ds(base, EW)])

    NQ = NCH // 8

    @functools.partial(
        pl.kernel,
        out_type=jax.ShapeDtypeStruct((NC, N, D), jnp.float32),
        mesh=mesh,
        compiler_params=pltpu.CompilerParams(use_tc_tiling_on_sc=False, needs_layout_passes=False),
        scratch_types=[
            pltpu.VMEM((8, 3, CH), jnp.int32),
            pltpu.VMEM((2, CH, D), jnp.float32),
            pltpu.VMEM((2, CH, D), jnp.float32),
            pltpu.VMEM_SHARED((N, D), jnp.float32),
            pltpu.SemaphoreType.DMA, pltpu.SemaphoreType.DMA,
            pltpu.SemaphoreType.DMA, pltpu.SemaphoreType.DMA,
            pltpu.SemaphoreType.DMA,
        ],
    )
    def sc_pass2(pk3_hbm, h_hbm, zeros_hbm, outp_hbm,
                 pk3_v, gb, sb, acc,
                 psm, gs0, gs1, ss0, ss1):
        gsem = (gs0, gs1)
        ssem = (ss0, ss1)
        cid = lax.axis_index("c")
        sid = lax.axis_index("s")
        wid = sid * NC + cid

        def start_g(k, b):
            return pltpu.async_copy(h_hbm.at[pk3_v.at[k, 0]], gb.at[b],
                                    gsem[b])

        def start_s(k, b):
            return pltpu.async_copy(sb.at[b], acc.at[pk3_v.at[k, 1]],
                                    ssem[b], add=True)

        with jax.named_scope("p2_zero"):
            pltpu.sync_copy(zeros_hbm.at[pl.ds(sid * RPT, RPT)],
                            acc.at[pl.ds(sid * RPT, RPT)])
            plsc.subcore_barrier()

        def qbody(q, _):
            c0 = q * 8
            pltpu.async_copy(pk3_hbm.at[wid, pl.ds(c0, 8)], pk3_v, psm).wait()
            gd = [None, None]
            sd = [None, None]
            gd[0] = start_g(0, 0)
            for k in range(8):
                b = k % 2
                if k + 1 < 8:
                    gd[1 - b] = start_g(k + 1, 1 - b)
                gd[b].wait()
                if sd[b] is not None:
                    sd[b].wait()

                @plsc.parallel_loop(0, CH, unroll=4)
                def _(r):
                    ci = plsc.load_gather(pk3_v.at[k, 2],
                                          [jnp.full((L,), 0, jnp.int32) + r])
                    cf = plsc.bitcast(ci, jnp.float32)
                    for j in range(D // L):
                        sb[b, r, pl.ds(j * L, L)] = (
                            gb[b, r, pl.ds(j * L, L)] * cf)

                sd[b] = start_s(k, b)
            sd[0].wait()
            sd[1].wait()
            return 0
        with jax.named_scope("p2_main"):
            lax.fori_loop(0, NQ, qbody, 0)
        with jax.named_scope("p2_out"):
            plsc.subcore_barrier()
            pltpu.sync_copy(acc.at[pl.ds(sid * RPT, RPT)],
                            outp_hbm.at[cid, pl.ds(sid * RPT, RPT)])

    return sc_pass1, sc_coef, sc_pass2


def _tc_call(body, out_shapes, *args):
    return pl.pallas_call(
        body,
        out_shape=out_shapes,
    )(*args)


def kernel(x, edge_index, W, a_src, a_dst, fc_w, fc_b):
    N, D = x.shape
    E = edge_index.shape[1]
    EW = -(-E // (NW * 8 * CH)) * (8 * CH)
    E_pad = EW * NW
    NCH = EW // CH

    src = edge_index[0]
    dst = edge_index[1]
    pad = E_pad - E
    spread = jnp.arange(pad, dtype=jnp.int32) % N
    src_p = jnp.concatenate([src, spread])
    dst_p = jnp.concatenate([dst, spread])
    src3 = src_p.reshape(NW, NCH, CH)
    dst3 = dst_p.reshape(NW, NCH, CH)
    zeros = jnp.zeros((N, D), jnp.float32)

    sc_pass1, sc_coef, sc_pass2 = _make_sc_kernels(N, D, E, E_pad)

    f32 = jnp.float32

    def layer(xin, first):
        if first:
            h, ssrc2, sdst2, cvec2 = _tc_call(
                _dense_body,
                (jax.ShapeDtypeStruct((N, D), f32),
                 jax.ShapeDtypeStruct((N, 1), f32),
                 jax.ShapeDtypeStruct((N, 1), f32),
                 jax.ShapeDtypeStruct((1, L), f32)),
                xin, W, a_src, a_dst)
            xn = xin
        else:
            x0, p0, p1 = xin
            xn, h, ssrc2, sdst2, cvec2 = _tc_call(
                _blend_dense_body,
                (jax.ShapeDtypeStruct((N, D), f32),
                 jax.ShapeDtypeStruct((N, D), f32),
                 jax.ShapeDtypeStruct((N, 1), f32),
                 jax.ShapeDtypeStruct((N, 1), f32),
                 jax.ShapeDtypeStruct((1, L), f32)),
                x0, p0, p1, W, a_src, a_dst)
        ssrc = ssrc2.reshape(N)
        sdst = sdst2.reshape(N)
        cvec = cvec2.reshape(L)
        dpart, ex = sc_pass1(src_p, dst_p, ssrc, sdst, cvec)
        rd2 = _tc_call(_rdenom_body,
                       jax.ShapeDtypeStruct((1, N), f32), dpart)
        rd = rd2.reshape(N)
        coef = sc_coef(dst_p, ex, rd)
        cf3 = jax.lax.bitcast_convert_type(coef, jnp.int32).reshape(NW, NCH, CH)
        pk3 = jnp.stack([src3, dst3, cf3], axis=2)
        outp = sc_pass2(pk3, h, zeros)
        return xn, outp

    _, p = layer(x, True)
    _, p2 = layer((x, p[0], p[1]), False)
    x2dummy, p3 = layer((x, p2[0], p2[1]), False)
    del x2dummy
    out, x3 = _tc_call(
        _final_body,
        (jax.ShapeDtypeStruct((N, fc_w.shape[1]), f32),
         jax.ShapeDtypeStruct((N, D), f32)),
        x, p3[0], p3[1], fc_w, fc_b)
    return (out, x3)


# speedup vs baseline: 1.1311x; 1.1311x over previous
import functools

import jax
import jax.numpy as jnp
from jax import lax
from jax.experimental import pallas as pl
from jax.experimental.pallas import tpu as pltpu
from jax.experimental.pallas import tpu_sc as plsc

NC, NS, L = 2, 16, 16
NW = NC * NS
BLEND = 0.5
NEG_SLOPE = 0.2
CH = 64


def _dense_body(x_ref, w_ref, asrc_ref, adst_ref,
                h_ref, ssrc_ref, sdst_ref, cvec_ref):
    h = jnp.dot(x_ref[...], w_ref[...], preferred_element_type=jnp.float32)
    h_ref[...] = h
    ssrc = jnp.sum(h * asrc_ref[...][None, :], axis=1, keepdims=True)
    sdst = jnp.sum(h * adst_ref[...][None, :], axis=1, keepdims=True)
    ssrc_ref[...] = ssrc
    sdst_ref[...] = sdst
    t = jnp.max(ssrc) + jnp.max(sdst)
    c = jnp.where(t > 0, t, NEG_SLOPE * t)
    cvec_ref[...] = jnp.full((1, L), c, jnp.float32)


def _blend_dense_body(x0_ref, p0_ref, p1_ref, w_ref, asrc_ref, adst_ref,
                      xn_ref, h_ref, ssrc_ref, sdst_ref, cvec_ref):
    xn = BLEND * (p0_ref[...] + p1_ref[...]) + (1.0 - BLEND) * x0_ref[...]
    xn_ref[...] = xn
    h = jnp.dot(xn, w_ref[...], preferred_element_type=jnp.float32)
    h_ref[...] = h
    ssrc = jnp.sum(h * asrc_ref[...][None, :], axis=1, keepdims=True)
    sdst = jnp.sum(h * adst_ref[...][None, :], axis=1, keepdims=True)
    ssrc_ref[...] = ssrc
    sdst_ref[...] = sdst
    t = jnp.max(ssrc) + jnp.max(sdst)
    c = jnp.where(t > 0, t, NEG_SLOPE * t)
    cvec_ref[...] = jnp.full((1, L), c, jnp.float32)


def _final_body(x0_ref, p0_ref, p1_ref, fcw_ref, fcb_ref, out_ref, x3_ref):
    x3 = BLEND * (p0_ref[...] + p1_ref[...]) + (1.0 - BLEND) * x0_ref[...]
    x3_ref[...] = x3
    out_ref[...] = (jnp.dot(x3, fcw_ref[...], preferred_element_type=jnp.float32)
                    + fcb_ref[...][None, :])


def _rdenom_body(dpart_ref, rd_ref):
    s = jnp.sum(dpart_ref[...], axis=0, keepdims=True)
    rd_ref[...] = 1.0 / (s + 1e-16)


def _make_sc_kernels(N, D, E, E_pad):
    EW = E_pad // NW
    G = EW // L
    NCH = EW // CH
    RPT = N // NS

    mesh = plsc.VectorSubcoreMesh(core_axis_name="c", subcore_axis_name="s",
                                  num_cores=NC, num_subcores=NS)

    @functools.partial(
        pl.kernel,
        out_type=(jax.ShapeDtypeStruct((NW, N), jnp.float32),
                  jax.ShapeDtypeStruct((E_pad,), jnp.float32)),
        mesh=mesh,
        compiler_params=pltpu.CompilerParams(use_tc_tiling_on_sc=False, needs_layout_passes=False),
        scratch_types=[
            pltpu.VMEM((EW,), jnp.int32),
            pltpu.VMEM((EW,), jnp.int32),
            pltpu.VMEM((N,), jnp.float32),
            pltpu.VMEM((N,), jnp.float32),
            pltpu.VMEM((N,), jnp.float32),
            pltpu.VMEM((EW,), jnp.float32),
            pltpu.VMEM((L,), jnp.float32),
        ],
    )
    def sc_pass1(src_hbm, dst_hbm, ssrc_hbm, sdst_hbm, cvec_hbm,
                 dpart_hbm, ex_hbm,
                 src_v, dst_v, ssrc_v, sdst_v, den_v, ex_v, cvec_v):
        cid = lax.axis_index("c")
        sid = lax.axis_index("s")
        wid = sid * NC + cid
        base = wid * EW
        pltpu.sync_copy(src_hbm.at[pl.ds(base, EW)], src_v)
        pltpu.sync_copy(dst_hbm.at[pl.ds(base, EW)], dst_v)
        pltpu.sync_copy(ssrc_hbm, ssrc_v)
        pltpu.sync_copy(sdst_hbm, sdst_v)
        pltpu.sync_copy(cvec_hbm, cvec_v)
        cv = cvec_v[...]

        def zbody(i, _):
            den_v[pl.ds(i * L, L)] = jnp.zeros((L,), jnp.float32)
            return 0
        lax.fori_loop(0, N // L, zbody, 0)

        lanes = lax.iota(jnp.int32, L)

        def gbody(g, _):
            off = g * L
            si = src_v[pl.ds(off, L)]
            di = dst_v[pl.ds(off, L)]
            a = plsc.load_gather(ssrc_v, [si])
            b = plsc.load_gather(sdst_v, [di])
            t = a + b
            t = jnp.where(t > 0, t, NEG_SLOPE * t)
            ex = jnp.exp(t - cv)
            gid = base + off + lanes
            ex = jnp.where(gid < E, ex, 0.0)
            ex_v[pl.ds(off, L)] = ex
            plsc.addupdate_scatter(den_v, [di], ex)
            return 0
        lax.fori_loop(0, G, gbody, 0)

        pltpu.sync_copy(den_v, dpart_hbm.at[wid])
        pltpu.sync_copy(ex_v, ex_hbm.at[pl.ds(base, EW)])

    @functools.partial(
        pl.kernel,
        out_type=jax.ShapeDtypeStruct((NW, NCH, CH), jnp.float32),
        mesh=mesh,
        compiler_params=pltpu.CompilerParams(use_tc_tiling_on_sc=False, needs_layout_passes=False),
        scratch_types=[
            pltpu.VMEM((NCH, CH), jnp.int32),
            pltpu.VMEM((EW,), jnp.float32),
            pltpu.VMEM((N,), jnp.float32),
            pltpu.VMEM((NCH, CH), jnp.float32),
        ],
    )
    def sc_coef(dst3_hbm, ex_hbm, rd_hbm, cf3_hbm, dst_v, ex_v, rd_v, cf_v):
        cid = lax.axis_index("c")
        sid = lax.axis_index("s")
        wid = sid * NC + cid
        base = wid * EW
        pltpu.sync_copy(dst3_hbm.at[wid], dst_v)
        pltpu.sync_copy(ex_hbm.at[pl.ds(base, EW)], ex_v)
        pltpu.sync_copy(rd_hbm, rd_v)

        def cbody(c, _):
            for u in range(CH // L):
                off = u * L
                di = dst_v[c, pl.ds(off, L)]
                cf_v[c, pl.ds(off, L)] = (
                    ex_v[pl.ds(c * CH + off, L)] * plsc.load_gather(rd_v, [di]))
            return 0
        lax.fori_loop(0, NCH, cbody, 0)
        pltpu.sync_copy(cf_v, cf3_hbm.at[wid])

    NQ = NCH // 8

    @functools.partial(
        pl.kernel,
        out_type=jax.ShapeDtypeStruct((NC, N, D), jnp.float32),
        mesh=mesh,
        compiler_params=pltpu.CompilerParams(use_tc_tiling_on_sc=False, needs_layout_passes=False),
        scratch_types=[
            pltpu.VMEM((8, CH), jnp.int32),
            pltpu.VMEM((8, CH), jnp.int32),
            pltpu.VMEM((8, CH), jnp.float32),
            pltpu.VMEM((25, D), jnp.float32),
            pltpu.VMEM((2, CH, D), jnp.float32),
            pltpu.VMEM((2, CH, D), jnp.float32),
            pltpu.VMEM_SHARED((N, D), jnp.float32),
            pltpu.SemaphoreType.DMA, pltpu.SemaphoreType.DMA,
            pltpu.SemaphoreType.DMA, pltpu.SemaphoreType.DMA,
            pltpu.SemaphoreType.DMA, pltpu.SemaphoreType.DMA,
            pltpu.SemaphoreType.DMA,
        ],
    )
    def sc_pass2(src3_hbm, dst3_hbm, cf3_hbm, h_hbm, outp_hbm,
                 sq_v, dq_v, cq_v, zbuf, gb, sb, acc,
                 psm0, psm1, psm2, gs0, gs1, ss0, ss1):
        gsem = (gs0, gs1)
        ssem = (ss0, ss1)
        cid = lax.axis_index("c")
        sid = lax.axis_index("s")
        wid = sid * NC + cid

        def start_g(k, b):
            return pltpu.async_copy(h_hbm.at[sq_v.at[k]], gb.at[b], gsem[b])

        def start_s(k, b):
            return pltpu.async_copy(sb.at[b], acc.at[dq_v.at[k]],
                                    ssem[b], add=True)

        with jax.named_scope("p2_zero"):
            def zb(i, _):
                for j in range(D // L):
                    zbuf[i, pl.ds(j * L, L)] = jnp.zeros((L,), jnp.float32)
                return 0
            lax.fori_loop(0, 25, zb, 0)
            def zcp(i, _):
                pltpu.sync_copy(zbuf, acc.at[pl.ds(sid * RPT + i * 25, 25)])
                return 0
            lax.fori_loop(0, RPT // 25, zcp, 0)
            plsc.subcore_barrier()

        def qbody(q, _):
            c0 = q * 8
            d0 = pltpu.async_copy(src3_hbm.at[wid, pl.ds(c0, 8)], sq_v, psm0)
            d1 = pltpu.async_copy(dst3_hbm.at[wid, pl.ds(c0, 8)], dq_v, psm1)
            d2 = pltpu.async_copy(cf3_hbm.at[wid, pl.ds(c0, 8)], cq_v, psm2)
            d0.wait()
            d1.wait()
            d2.wait()
            gd = [None, None]
            sd = [None, None]
            gd[0] = start_g(0, 0)
            for k in range(8):
                b = k % 2
                if k + 1 < 8:
                    gd[1 - b] = start_g(k + 1, 1 - b)
                gd[b].wait()
                if sd[b] is not None:
                    sd[b].wait()

                @plsc.parallel_loop(0, CH, unroll=2)
                def _(r):
                    cf = plsc.load_gather(cq_v.at[k],
                                          [jnp.full((L,), 0, jnp.int32) + r])
                    for j in range(D // L):
                        sb[b, r, pl.ds(j * L, L)] = (
                            gb[b, r, pl.ds(j * L, L)] * cf)

                sd[b] = start_s(k, b)
            sd[0].wait()
            sd[1].wait()
            return 0
        with jax.named_scope("p2_main"):
            lax.fori_loop(0, NQ, qbody, 0)
        with jax.named_scope("p2_out"):
            plsc.subcore_barrier()
            pltpu.sync_copy(acc.at[pl.ds(sid * RPT, RPT)],
                            outp_hbm.at[cid, pl.ds(sid * RPT, RPT)])

    return sc_pass1, sc_coef, sc_pass2


def _tc_call(body, out_shapes, *args):
    return pl.pallas_call(
        body,
        out_shape=out_shapes,
    )(*args)


def kernel(x, edge_index, W, a_src, a_dst, fc_w, fc_b):
    N, D = x.shape
    E = edge_index.shape[1]
    EW = -(-E // (NW * 8 * CH)) * (8 * CH)
    E_pad = EW * NW
    NCH = EW // CH

    src = edge_index[0]
    dst = edge_index[1]
    pad = E_pad - E
    spread = jnp.arange(pad, dtype=jnp.int32) % N
    src_p = jnp.concatenate([src, spread])
    dst_p = jnp.concatenate([dst, spread])
    src3 = src_p.reshape(NW, NCH, CH)
    dst3 = dst_p.reshape(NW, NCH, CH)

    sc_pass1, sc_coef, sc_pass2 = _make_sc_kernels(N, D, E, E_pad)

    f32 = jnp.float32

    def layer(xin, first):
        if first:
            h, ssrc2, sdst2, cvec2 = _tc_call(
                _dense_body,
                (jax.ShapeDtypeStruct((N, D), f32),
                 jax.ShapeDtypeStruct((N, 1), f32),
                 jax.ShapeDtypeStruct((N, 1), f32),
                 jax.ShapeDtypeStruct((1, L), f32)),
                xin, W, a_src, a_dst)
            xn = xin
        else:
            x0, p0, p1 = xin
            xn, h, ssrc2, sdst2, cvec2 = _tc_call(
                _blend_dense_body,
                (jax.ShapeDtypeStruct((N, D), f32),
                 jax.ShapeDtypeStruct((N, D), f32),
                 jax.ShapeDtypeStruct((N, 1), f32),
                 jax.ShapeDtypeStruct((N, 1), f32),
                 jax.ShapeDtypeStruct((1, L), f32)),
                x0, p0, p1, W, a_src, a_dst)
        ssrc = ssrc2.reshape(N)
        sdst = sdst2.reshape(N)
        cvec = cvec2.reshape(L)
        dpart, ex = sc_pass1(src_p, dst_p, ssrc, sdst, cvec)
        rd2 = _tc_call(_rdenom_body,
                       jax.ShapeDtypeStruct((1, N), f32), dpart)
        rd = rd2.reshape(N)
        cf3 = sc_coef(dst3, ex, rd)
        outp = sc_pass2(src3, dst3, cf3, h)
        return xn, outp

    _, p = layer(x, True)
    _, p2 = layer((x, p[0], p[1]), False)
    x2dummy, p3 = layer((x, p2[0], p2[1]), False)
    del x2dummy
    out, x3 = _tc_call(
        _final_body,
        (jax.ShapeDtypeStruct((N, fc_w.shape[1]), f32),
         jax.ShapeDtypeStruct((N, D), f32)),
        x, p3[0], p3[1], fc_w, fc_b)
    return (out, x3)


# speedup vs baseline: 1.1908x; 1.0528x over previous
import functools

import jax
import jax.numpy as jnp
from jax import lax
from jax.experimental import pallas as pl
from jax.experimental.pallas import tpu as pltpu
from jax.experimental.pallas import tpu_sc as plsc

NC, NS, L = 2, 16, 16
NW = NC * NS
BLEND = 0.5
NEG_SLOPE = 0.2
CH = 64


def _dense_body(x_ref, w_ref, asrc_ref, adst_ref,
                h_ref, ssrc_ref, sdst_ref, cvec_ref):
    h = jnp.dot(x_ref[...], w_ref[...], preferred_element_type=jnp.float32)
    h_ref[...] = h
    ssrc = jnp.sum(h * asrc_ref[...][None, :], axis=1, keepdims=True)
    sdst = jnp.sum(h * adst_ref[...][None, :], axis=1, keepdims=True)
    ssrc_ref[...] = ssrc
    sdst_ref[...] = sdst
    t = jnp.max(ssrc) + jnp.max(sdst)
    c = jnp.where(t > 0, t, NEG_SLOPE * t)
    cvec_ref[...] = jnp.full((1, L), c, jnp.float32)


def _blend_dense_body(x0_ref, p0_ref, p1_ref, w_ref, asrc_ref, adst_ref,
                      xn_ref, h_ref, ssrc_ref, sdst_ref, cvec_ref):
    xn = BLEND * (p0_ref[...] + p1_ref[...]) + (1.0 - BLEND) * x0_ref[...]
    xn_ref[...] = xn
    h = jnp.dot(xn, w_ref[...], preferred_element_type=jnp.float32)
    h_ref[...] = h
    ssrc = jnp.sum(h * asrc_ref[...][None, :], axis=1, keepdims=True)
    sdst = jnp.sum(h * adst_ref[...][None, :], axis=1, keepdims=True)
    ssrc_ref[...] = ssrc
    sdst_ref[...] = sdst
    t = jnp.max(ssrc) + jnp.max(sdst)
    c = jnp.where(t > 0, t, NEG_SLOPE * t)
    cvec_ref[...] = jnp.full((1, L), c, jnp.float32)


def _final_body(x0_ref, p0_ref, p1_ref, fcw_ref, fcb_ref, out_ref, x3_ref):
    x3 = BLEND * (p0_ref[...] + p1_ref[...]) + (1.0 - BLEND) * x0_ref[...]
    x3_ref[...] = x3
    out_ref[...] = (jnp.dot(x3, fcw_ref[...], preferred_element_type=jnp.float32)
                    + fcb_ref[...][None, :])


def _rdenom_body(dpart_ref, rd_ref):
    s = jnp.sum(dpart_ref[...], axis=0, keepdims=True)
    rd_ref[...] = 1.0 / (s + 1e-16)


def _make_sc_kernels(N, D, E, E_pad):
    EW = E_pad // NW
    G = EW // L
    NCH = EW // CH
    RPT = N // NS

    mesh = plsc.VectorSubcoreMesh(core_axis_name="c", subcore_axis_name="s",
                                  num_cores=NC, num_subcores=NS)

    @functools.partial(
        pl.kernel,
        out_type=(jax.ShapeDtypeStruct((NW, N), jnp.float32),
                  jax.ShapeDtypeStruct((E_pad,), jnp.float32)),
        mesh=mesh,
        compiler_params=pltpu.CompilerParams(use_tc_tiling_on_sc=False, needs_layout_passes=False),
        scratch_types=[
            pltpu.VMEM((EW,), jnp.int32),
            pltpu.VMEM((EW,), jnp.int32),
            pltpu.VMEM((N,), jnp.float32),
            pltpu.VMEM((N,), jnp.float32),
            pltpu.VMEM((N,), jnp.float32),
            pltpu.VMEM((EW,), jnp.float32),
            pltpu.VMEM((L,), jnp.float32),
        ],
    )
    def sc_pass1(src_hbm, dst_hbm, ssrc_hbm, sdst_hbm, cvec_hbm,
                 dpart_hbm, ex_hbm,
                 src_v, dst_v, ssrc_v, sdst_v, den_v, ex_v, cvec_v):
        cid = lax.axis_index("c")
        sid = lax.axis_index("s")
        wid = sid * NC + cid
        base = wid * EW
        pltpu.sync_copy(src_hbm.at[pl.ds(base, EW)], src_v)
        pltpu.sync_copy(dst_hbm.at[pl.ds(base, EW)], dst_v)
        pltpu.sync_copy(ssrc_hbm, ssrc_v)
        pltpu.sync_copy(sdst_hbm, sdst_v)
        pltpu.sync_copy(cvec_hbm, cvec_v)
        cv = cvec_v[...]

        def zbody(i, _):
            den_v[pl.ds(i * L, L)] = jnp.zeros((L,), jnp.float32)
            return 0
        lax.fori_loop(0, N // L, zbody, 0)

        lanes = lax.iota(jnp.int32, L)

        def gbody(g, _):
            off = g * L
            si = src_v[pl.ds(off, L)]
            di = dst_v[pl.ds(off, L)]
            a = plsc.load_gather(ssrc_v, [si])
            b = plsc.load_gather(sdst_v, [di])
            t = a + b
            t = jnp.where(t > 0, t, NEG_SLOPE * t)
            ex = jnp.exp(t - cv)
            gid = base + off + lanes
            ex = jnp.where(gid < E, ex, 0.0)
            ex_v[pl.ds(off, L)] = ex
            plsc.addupdate_scatter(den_v, [di], ex)
            return 0
        lax.fori_loop(0, G, gbody, 0)

        pltpu.sync_copy(den_v, dpart_hbm.at[wid])
        pltpu.sync_copy(ex_v, ex_hbm.at[pl.ds(base, EW)])

    NQ = NCH // 8

    @functools.partial(
        pl.kernel,
        out_type=jax.ShapeDtypeStruct((NC, N, D), jnp.float32),
        mesh=mesh,
        compiler_params=pltpu.CompilerParams(use_tc_tiling_on_sc=False, needs_layout_passes=False),
        scratch_types=[
            pltpu.VMEM((8, CH), jnp.int32),
            pltpu.VMEM((8, CH), jnp.int32),
            pltpu.VMEM((8 * CH,), jnp.float32),
            pltpu.VMEM((N,), jnp.float32),
            pltpu.VMEM((25, D), jnp.float32),
            pltpu.VMEM((2, CH, D), jnp.float32),
            pltpu.VMEM((2, CH, D), jnp.float32),
            pltpu.VMEM_SHARED((N, D), jnp.float32),
            pltpu.SemaphoreType.DMA, pltpu.SemaphoreType.DMA,
            pltpu.SemaphoreType.DMA, pltpu.SemaphoreType.DMA,
            pltpu.SemaphoreType.DMA, pltpu.SemaphoreType.DMA,
            pltpu.SemaphoreType.DMA,
        ],
    )
    def sc_pass2(src3_hbm, dst3_hbm, ex_hbm, rd_hbm, h_hbm, outp_hbm,
                 sq_v, dq_v, cq_v, rd_v, zbuf, gb, sb, acc,
                 psm0, psm1, psm2, gs0, gs1, ss0, ss1):
        gsem = (gs0, gs1)
        ssem = (ss0, ss1)
        cid = lax.axis_index("c")
        sid = lax.axis_index("s")
        wid = sid * NC + cid
        base = wid * EW
        pltpu.sync_copy(rd_hbm, rd_v)

        def start_g(k, b):
            return pltpu.async_copy(h_hbm.at[sq_v.at[k]], gb.at[b], gsem[b])

        def start_s(k, b):
            return pltpu.async_copy(sb.at[b], acc.at[dq_v.at[k]],
                                    ssem[b], add=True)

        with jax.named_scope("p2_zero"):
            def zb(i, _):
                for j in range(D // L):
                    zbuf[i, pl.ds(j * L, L)] = jnp.zeros((L,), jnp.float32)
                return 0
            lax.fori_loop(0, 25, zb, 0)
            def zcp(i, _):
                pltpu.sync_copy(zbuf, acc.at[pl.ds(sid * RPT + i * 25, 25)])
                return 0
            lax.fori_loop(0, RPT // 25, zcp, 0)
            plsc.subcore_barrier()

        def qbody(q, _):
            c0 = q * 8
            d0 = pltpu.async_copy(src3_hbm.at[wid, pl.ds(c0, 8)], sq_v, psm0)
            d1 = pltpu.async_copy(dst3_hbm.at[wid, pl.ds(c0, 8)], dq_v, psm1)
            d2 = pltpu.async_copy(ex_hbm.at[pl.ds(base + c0 * CH, 8 * CH)],
                                  cq_v, psm2)
            d0.wait()
            d1.wait()
            d2.wait()
            gd = [None, None]
            sd = [None, None]
            gd[0] = start_g(0, 0)
            for kk in range(8):
                for u in range(CH // L):
                    off = kk * CH + u * L
                    di = dq_v[kk, pl.ds(u * L, L)]
                    cq_v[pl.ds(off, L)] = (
                        cq_v[pl.ds(off, L)] * plsc.load_gather(rd_v, [di]))
            for k in range(8):
                b = k % 2
                if k + 1 < 8:
                    gd[1 - b] = start_g(k + 1, 1 - b)
                gd[b].wait()
                if sd[b] is not None:
                    sd[b].wait()

                @plsc.parallel_loop(0, CH, unroll=2)
                def _(r):
                    cf = plsc.load_gather(cq_v,
                                          [jnp.full((L,), k * CH, jnp.int32) + r])
                    for j in range(D // L):
                        sb[b, r, pl.ds(j * L, L)] = (
                            gb[b, r, pl.ds(j * L, L)] * cf)

                sd[b] = start_s(k, b)
            sd[0].wait()
            sd[1].wait()
            return 0
        with jax.named_scope("p2_main"):
            lax.fori_loop(0, NQ, qbody, 0)
        with jax.named_scope("p2_out"):
            plsc.subcore_barrier()
            pltpu.sync_copy(acc.at[pl.ds(sid * RPT, RPT)],
                            outp_hbm.at[cid, pl.ds(sid * RPT, RPT)])

    return sc_pass1, sc_pass2


def _tc_call(body, out_shapes, *args):
    return pl.pallas_call(
        body,
        out_shape=out_shapes,
    )(*args)


def kernel(x, edge_index, W, a_src, a_dst, fc_w, fc_b):
    N, D = x.shape
    E = edge_index.shape[1]
    EW = -(-E // (NW * 8 * CH)) * (8 * CH)
    E_pad = EW * NW
    NCH = EW // CH

    src = edge_index[0]
    dst = edge_index[1]
    pad = E_pad - E
    spread = jnp.arange(pad, dtype=jnp.int32) % N
    src_p = jnp.concatenate([src, spread])
    dst_p = jnp.concatenate([dst, spread])
    src3 = src_p.reshape(NW, NCH, CH)
    dst3 = dst_p.reshape(NW, NCH, CH)

    sc_pass1, sc_pass2 = _make_sc_kernels(N, D, E, E_pad)

    f32 = jnp.float32

    def layer(xin, first):
        if first:
            h, ssrc2, sdst2, cvec2 = _tc_call(
                _dense_body,
                (jax.ShapeDtypeStruct((N, D), f32),
                 jax.ShapeDtypeStruct((N, 1), f32),
                 jax.ShapeDtypeStruct((N, 1), f32),
                 jax.ShapeDtypeStruct((1, L), f32)),
                xin, W, a_src, a_dst)
            xn = xin
        else:
            x0, p0, p1 = xin
            xn, h, ssrc2, sdst2, cvec2 = _tc_call(
                _blend_dense_body,
                (jax.ShapeDtypeStruct((N, D), f32),
                 jax.ShapeDtypeStruct((N, D), f32),
                 jax.ShapeDtypeStruct((N, 1), f32),
                 jax.ShapeDtypeStruct((N, 1), f32),
                 jax.ShapeDtypeStruct((1, L), f32)),
                x0, p0, p1, W, a_src, a_dst)
        ssrc = ssrc2.reshape(N)
        sdst = sdst2.reshape(N)
        cvec = cvec2.reshape(L)
        dpart, ex = sc_pass1(src_p, dst_p, ssrc, sdst, cvec)
        rd2 = _tc_call(_rdenom_body,
                       jax.ShapeDtypeStruct((1, N), f32), dpart)
        rd = rd2.reshape(N)
        outp = sc_pass2(src3, dst3, ex, rd, h)
        return xn, outp

    _, p = layer(x, True)
    _, p2 = layer((x, p[0], p[1]), False)
    x2dummy, p3 = layer((x, p2[0], p2[1]), False)
    del x2dummy
    out, x3 = _tc_call(
        _final_body,
        (jax.ShapeDtypeStruct((N, fc_w.shape[1]), f32),
         jax.ShapeDtypeStruct((N, D), f32)),
        x, p3[0], p3[1], fc_w, fc_b)
    return (out, x3)
